# Initial kernel scaffold; baseline (speedup 1.0000x reference)
import jax
import jax.numpy as jnp
from jax.experimental import pallas as pl


def kernel(boxes, scores):
    raise NotImplementedError("write your pallas kernel here")


# speedup vs baseline: 20.5183x; 20.5183x over previous
import functools

import jax
import jax.numpy as jnp
from jax import lax
from jax.experimental import pallas as pl
from jax.experimental.pallas import tpu as pltpu
from jax.experimental.pallas import tpu_sc as plsc

N = 20000
NPAD = 20480
NW = 32
L = 16
TILE = 64
NTILES = NPAD // TILE
ECAP = 2048
CCAP = 8192
KOUT = 1000
THR = 0.7

_mesh = plsc.VectorSubcoreMesh(core_axis_name="c", subcore_axis_name="s")
_params = pltpu.CompilerParams(needs_layout_passes=False)


def _wid():
    return lax.axis_index("s") * 2 + lax.axis_index("c")


def _iota():
    return lax.broadcasted_iota(jnp.int32, (L,), 0)


def _bcast(vec, lane):
    idx = jnp.full((L,), lane, jnp.int32)
    return vec.at[idx].get(mode="promise_in_bounds")


def _prefix(m_i32, iota):
    pc = m_i32
    for sh in (1, 2, 4, 8):
        src = jnp.maximum(iota - sh, 0)
        shifted = pc.at[src].get(mode="promise_in_bounds")
        pc = pc + jnp.where(iota >= sh, shifted, 0)
    return pc


def _anylane(mask, iota):
    m = jnp.where(mask, 1, 0)
    for sh in (8, 4, 2, 1):
        m = jnp.maximum(m, m.at[iota ^ sh].get(mode="promise_in_bounds"))
    return m[0]


def _gather_body(x1u, y1u, x2u, y2u, order_h, sx1, sy1, sx2, sy2, idxv, buf, sem):
    w = _wid()
    base = w * (NPAD // NW)
    srcs = (x1u, y1u, x2u, y2u)
    dsts = (sx1, sy1, sx2, sy2)
    for c in range(5):
        off = base + 128 * c
        pltpu.sync_copy(order_h.at[pl.ds(off, 128)], idxv)
        for s, d in zip(srcs, dsts):
            pltpu.async_copy(s.at[idxv], buf, sem).wait()
            pltpu.sync_copy(buf, d.at[pl.ds(off, 128)])


@jax.jit
def _gather_call(x1u, y1u, x2u, y2u, order):
    f = functools.partial(
        pl.kernel,
        out_type=[jax.ShapeDtypeStruct((NPAD,), jnp.float32)] * 4,
        mesh=_mesh,
        compiler_params=_params,
        scratch_types=[
            pltpu.VMEM((128,), jnp.int32),
            pltpu.VMEM((128,), jnp.float32),
            pltpu.SemaphoreType.DMA,
        ],
    )(_gather_body)
    return f(x1u, y1u, x2u, y2u, order)


def _probe_body(x_h, out_h, xv, tmp, ov):
    w = _wid()
    iota = _iota()
    pltpu.sync_copy(x_h, xv)
    a = xv[pl.ds(0, L)]
    mask = a > 0.5
    pc = _prefix(jnp.where(mask, 1, 0), iota)
    ps = _bcast(pc, L - 1)
    tmp[...] = ps
    s = ps[0]

    def yes(o):
        return o + _bcast(a, 3).astype(jnp.int32)[0]

    o = lax.cond(s > 0, yes, lambda o: o, jnp.int32(0))
    idx = jnp.clip(o + pc - 1, 0, L - 1)
    plsc.store_scatter(ov, [idx], pc + s, mask=mask)
    g = plsc.load_gather(ov, [iota], mask=mask)
    plsc.addupdate_scatter(ov, [idx], g, mask=mask)

    @pl.when(w == 0)
    def _():
        pltpu.sync_copy(ov, out_h)


@jax.jit
def _probe_call(x):
    f = functools.partial(
        pl.kernel,
        out_type=jax.ShapeDtypeStruct((L,), jnp.int32),
        mesh=_mesh,
        compiler_params=_params,
        scratch_types=[
            pltpu.VMEM((L,), jnp.float32),
            pltpu.VMEM((L,), jnp.int32),
            pltpu.VMEM((L,), jnp.int32),
        ],
    )(_probe_body)
    return f(x)


def _pair_body(sx1_h, sy1_h, sx2_h, sy2_h, ebuf_h, cnt_h, x1v, y1v, x2v, y2v,
               ebuf, cntv):
    w = _wid()
    pltpu.sync_copy(sx1_h, x1v)
    pltpu.sync_copy(sy1_h, y1v)
    pltpu.sync_copy(sx2_h, x2v)
    pltpu.sync_copy(sy2_h, y2v)
    iota = _iota()

    def tile_body(q, off):
        m = q // 2
        tid = jnp.where(q % 2 == 0, 64 * m + w, 64 * m + 63 - w)
        tbase = tid * TILE
        cx1 = [x1v[pl.ds(tbase + L * v, L)] for v in range(4)]
        cy1 = [y1v[pl.ds(tbase + L * v, L)] for v in range(4)]
        cx2 = [x2v[pl.ds(tbase + L * v, L)] for v in range(4)]
        cy2 = [y2v[pl.ds(tbase + L * v, L)] for v in range(4)]
        car = [jnp.abs((cx1[v] - cx2[v]) * (cy1[v] - cy2[v])) for v in range(4)]
        cpos = [tbase + L * v + iota for v in range(4)]

        def row_geom(bx1, by1, bx2, by2, rr):
            rx1 = _bcast(bx1, rr)
            ry1 = _bcast(by1, rr)
            rx2 = _bcast(bx2, rr)
            ry2 = _bcast(by2, rr)
            rar = jnp.abs((rx1 - rx2) * (ry1 - ry2))
            return rx1, ry1, rx2, ry2, rar

        def pair_parts(rg, v):
            rx1, ry1, rx2, ry2, rar = rg
            xl = jnp.maximum(rx1, cx1[v])
            yb = jnp.maximum(ry1, cy1[v])
            xr = jnp.minimum(rx2, cx2[v])
            yt = jnp.minimum(ry2, cy2[v])
            valid = (xl <= xr) & (yb <= yt)
            inter = (xl - xr) * (yb - yt)
            denom = (rar + car[v]) - inter
            return valid, inter, denom

        def block_body(rb, off):
            rbase = rb * L
            bx1 = x1v[pl.ds(rbase, L)]
            by1 = y1v[pl.ds(rbase, L)]
            bx2 = x2v[pl.ds(rbase, L)]
            by2 = y2v[pl.ds(rbase, L)]

            def group(off, r0):
                acc = jnp.zeros((L,), jnp.bool_)
                for rr in range(r0, r0 + 8):
                    rg = row_geom(bx1, by1, bx2, by2, rr)
                    rpos = rbase + rr
                    for v in range(4):
                        valid, inter, denom = pair_parts(rg, v)
                        hit = valid & (inter * 1.00002 > THR * denom)
                        hit = hit & (cpos[v] > rpos)
                        acc = acc | hit
                def rec(o):
                    for rr in range(r0, r0 + 8):
                        rg = row_geom(bx1, by1, bx2, by2, rr)
                        rpos = rbase + rr
                        for v in range(4):
                            valid, inter, denom = pair_parts(rg, v)
                            iou = jnp.where(valid, inter / denom, 0.0)
                            hit = (iou > THR) & (cpos[v] > rpos)
                            pc = _prefix(jnp.where(hit, 1, 0), iota)
                            tot = _bcast(pc, L - 1)[0]
                            idx = jnp.clip(o + pc - 1, 0, ECAP - 1)
                            val = (rpos << 15) | cpos[v]
                            plsc.store_scatter(ebuf, [idx], val,
                                               mask=hit)
                            o = jnp.minimum(o + tot, ECAP)
                    return o
                return lax.cond(_anylane(acc, iota) > 0, rec,
                                lambda o: o, off)

            off = group(off, 0)
            off = group(off, 8)
            return off

        nblocks = 4 * tid + 4
        return lax.fori_loop(0, nblocks, block_body, off)

    off = lax.fori_loop(0, NTILES // NW, tile_body, jnp.zeros((L,), jnp.int32))
    cntv[...] = jnp.where(iota == 0, off, 0)
    pltpu.sync_copy(ebuf, ebuf_h.at[pl.ds(w * ECAP, ECAP)])
    pltpu.sync_copy(cntv, cnt_h.at[pl.ds(w * L, L)])


@jax.jit
def _pair_call(sx1, sy1, sx2, sy2):
    f = functools.partial(
        pl.kernel,
        out_type=[
            jax.ShapeDtypeStruct((NW * ECAP,), jnp.int32),
            jax.ShapeDtypeStruct((NW * L,), jnp.int32),
        ],
        mesh=_mesh,
        compiler_params=_params,
        scratch_types=[
            pltpu.VMEM((NPAD,), jnp.float32),
            pltpu.VMEM((NPAD,), jnp.float32),
            pltpu.VMEM((NPAD,), jnp.float32),
            pltpu.VMEM((NPAD,), jnp.float32),
            pltpu.VMEM((ECAP,), jnp.int32),
            pltpu.VMEM((L,), jnp.int32),
        ],
    )(_pair_body)
    return f(sx1, sy1, sx2, sy2)


def _resolve_body(ebuf_h, cnt_h, order_h, kidx_h, keep_h, stag, comb, cntv,
                  keep, supp, korig, ochunk, oidx, chgv):
    w = _wid()
    iota = _iota()

    @pl.when(w == 0)
    def _():
        pltpu.sync_copy(cnt_h, cntv)

        def compact_w(wi, etot):
            pltpu.sync_copy(ebuf_h.at[pl.ds(wi * ECAP, ECAP)], stag)
            cw = cntv[pl.ds(wi * L, L)][0]

            def vbody(v, eo):
                e = stag[pl.ds(v * L, L)]
                msk = (v * L + iota) < cw
                pc = _prefix(jnp.where(msk, 1, 0), iota)
                idx = jnp.clip(eo + pc - 1, 0, CCAP - 1)
                plsc.store_scatter(comb, [idx], e, mask=msk)
                return jnp.minimum(eo + _bcast(pc, L - 1)[0], CCAP)

            return lax.fori_loop(0, (cw + L - 1) // L, vbody, etot)

        etot = lax.fori_loop(0, NW, compact_w, jnp.int32(0))
        nev = (etot + L - 1) // L

        def initk(p, _):
            keep[pl.ds(p * L, L)] = jnp.ones((L,), jnp.float32)
            return 0
        lax.fori_loop(0, NPAD // L, initk, 0)

        def fp_cond(c):
            changed, it = c
            return changed & (it < 1024)

        def fp_body(c):
            _, it = c

            def zs(p, _):
                supp[pl.ds(p * L, L)] = jnp.zeros((L,), jnp.float32)
                return 0
            lax.fori_loop(0, NPAD // L, zs, 0)

            def ev(v, _):
                e = comb[pl.ds(v * L, L)]
                msk = (v * L + iota) < etot
                ei = jnp.where(msk, e >> 15, 0)
                ej = jnp.where(msk, e & 32767, 0)
                ki = plsc.load_gather(keep, [ei], mask=msk)
                plsc.addupdate_scatter(supp, [ej], ki, mask=msk)
                return 0
            lax.fori_loop(0, nev, ev, 0)

            chgv[...] = jnp.zeros((L,), jnp.float32)

            def rb(p, _):
                sl = pl.ds(p * L, L)
                kn = jnp.where(supp[sl] > 0.0, 0.0, 1.0)
                chgv[...] = chgv[...] + jnp.abs(kn - keep[sl])
                keep[sl] = kn
                return 0
            lax.fori_loop(0, NPAD // L, rb, 0)
            return _anylane(chgv[...] > 0.0, iota) > 0, it + 1

        lax.while_loop(fp_cond, fp_body, (jnp.bool_(True), jnp.int32(0)))

        def chunk_body(c, off):
            pltpu.sync_copy(order_h.at[pl.ds(c * 2048, 2048)], ochunk)

            def vbody(v, off):
                p0 = c * 2048 + v * L
                ov = ochunk[pl.ds(v * L, L)]
                kv = keep[pl.ds(p0, L)]
                real = (p0 + iota) < N
                kept = (kv > 0.0) & real
                plsc.store_scatter(korig, [ov],
                                   jnp.where(kept, 1, 0), mask=real)
                pc = _prefix(jnp.where(kept, 1, 0), iota)
                idx = off + pc - 1
                wm = kept & (idx < KOUT)
                plsc.store_scatter(oidx, [jnp.clip(idx, 0, KOUT - 1)],
                                   ov, mask=wm)
                return off + _bcast(pc, L - 1)[0]

            return lax.fori_loop(0, 128, vbody, off)

        off = lax.fori_loop(0, NPAD // 2048, chunk_body, jnp.int32(0))

        def pad_fill(off):
            def vbody(v, off):
                kv = korig[pl.ds(v * L, L)]
                gone = kv == 0
                pc = _prefix(jnp.where(gone, 1, 0), iota)
                idx = off + pc - 1
                wm = gone & (idx < KOUT)
                plsc.store_scatter(oidx, [jnp.clip(idx, 0, KOUT - 1)],
                                   v * L + iota, mask=wm)
                return off + _bcast(pc, L - 1)[0]
            return lax.fori_loop(0, N // L, vbody, off)

        lax.cond(off < KOUT, pad_fill, lambda o: o, off)

        pltpu.sync_copy(oidx, kidx_h)
        pltpu.sync_copy(korig, keep_h)


@jax.jit
def _resolve_call(ebuf, cnt, order):
    f = functools.partial(
        pl.kernel,
        out_type=[
            jax.ShapeDtypeStruct((1024,), jnp.int32),
            jax.ShapeDtypeStruct((N,), jnp.int32),
        ],
        compiler_params=_params,
        mesh=_mesh,
        scratch_types=[
            pltpu.VMEM((ECAP,), jnp.int32),
            pltpu.VMEM((CCAP,), jnp.int32),
            pltpu.VMEM((NW * L,), jnp.int32),
            pltpu.VMEM((NPAD,), jnp.float32),
            pltpu.VMEM((NPAD,), jnp.float32),
            pltpu.VMEM((N,), jnp.int32),
            pltpu.VMEM((2048,), jnp.int32),
            pltpu.VMEM((1024,), jnp.int32),
            pltpu.VMEM((L,), jnp.float32),
        ],
    )(_resolve_body)
    return f(ebuf, cnt, order)


def kernel(boxes, scores):
    boxes = boxes.astype(jnp.float32)
    scores = scores.astype(jnp.float32)
    npad = NPAD - N
    pk = jnp.arange(npad, dtype=jnp.float32)
    padboxes = jnp.stack([
        2e6 + 1000.0 * pk,
        jnp.full((npad,), 2e6, jnp.float32),
        2e6 + 1000.0 * pk + 10.0,
        jnp.full((npad,), 2e6 + 10.0, jnp.float32),
    ], axis=1)
    boxes_p = jnp.concatenate([boxes, padboxes], axis=0)
    scores_p = jnp.concatenate(
        [scores, jnp.full((npad,), -1.0, jnp.float32)], axis=0)
    order = jnp.argsort(-scores_p).astype(jnp.int32)
    sx1, sy1, sx2, sy2 = _gather_call(
        boxes_p[:, 0], boxes_p[:, 1], boxes_p[:, 2], boxes_p[:, 3], order)
    ebuf, cnt = _pair_call(sx1, sy1, sx2, sy2)
    kidx, keepi = _resolve_call(ebuf, cnt, order)
    return kidx[:KOUT], keepi.astype(jnp.bool_)
